# baseline (speedup 1.0000x reference)
import functools

import jax
import jax.numpy as jnp
from jax import lax
from jax.experimental import pallas as pl
from jax.experimental.pallas import tpu as pltpu
from jax.experimental.pallas import tpu_sc as plsc

NC, NS = 2, 16
NW = NC * NS
DP = 128
CHUNK = 128
NBUF = 4


@functools.partial(jax.jit, static_argnames=("total",))
def _gather(idx_flat, table, total):
    b_per_w = total // NW
    n_chunks = b_per_w // CHUNK
    n_waves = n_chunks // NBUF
    mesh = plsc.VectorSubcoreMesh(
        core_axis_name="c", subcore_axis_name="s", num_cores=NC, num_subcores=NS
    )

    @functools.partial(
        pl.kernel,
        mesh=mesh,
        out_type=jax.ShapeDtypeStruct((total, DP), jnp.float32),
        scratch_types=[
            pltpu.VMEM((b_per_w,), jnp.int32),
            pltpu.VMEM((NBUF, CHUNK, DP), jnp.float32),
            pltpu.SemaphoreType.DMA,
            pltpu.SemaphoreType.DMA,
            pltpu.SemaphoreType.DMA,
            pltpu.SemaphoreType.DMA,
            pltpu.SemaphoreType.DMA,
            pltpu.SemaphoreType.DMA,
            pltpu.SemaphoreType.DMA,
            pltpu.SemaphoreType.DMA,
        ],
        compiler_params=pltpu.CompilerParams(use_tc_tiling_on_sc=False),
    )
    def kern(idx_hbm, table_hbm, out_hbm, idx_v, rows_v, g0, g1, g2, g3, o0, o1, o2, o3):
        gsem = (g0, g1, g2, g3)
        osem = (o0, o1, o2, o3)
        wid = lax.axis_index("s") * NC + lax.axis_index("c")
        base = wid * b_per_w
        pltpu.sync_copy(idx_hbm.at[pl.ds(base, b_per_w)], idx_v)

        def gather_desc(c, b):
            return pltpu.make_async_copy(
                table_hbm.at[idx_v.at[pl.ds(c * CHUNK, CHUNK)]],
                rows_v.at[b],
                gsem[b],
            )

        def out_desc(c, b):
            return pltpu.make_async_copy(
                rows_v.at[b],
                out_hbm.at[pl.ds(base + c * CHUNK, CHUNK)],
                osem[b],
            )

        for b in range(NBUF):
            gather_desc(b, b).start()

        @pl.loop(0, n_waves - 1)
        def _(p):
            c = p * NBUF
            for b in range(NBUF):
                gather_desc(c + b, b).wait()
                out_desc(c + b, b).start()
            for b in range(NBUF):
                out_desc(c + b, b).wait()
                gather_desc(c + NBUF + b, b).start()

        c_last = (n_waves - 1) * NBUF
        for b in range(NBUF):
            gather_desc(c_last + b, b).wait()
            out_desc(c_last + b, b).start()
        for b in range(NBUF):
            out_desc(c_last + b, b).wait()

    return kern(idx_flat, table)


def kernel(x, weight):
    bsz, seq = x.shape
    seq_p = 56
    xp = jnp.pad(x.astype(jnp.int32), ((0, 0), (0, seq_p - seq)))
    idx_flat = xp.reshape(bsz * seq_p)
    table = jnp.pad(weight, ((0, 0), (0, DP - weight.shape[1])))
    out = _gather(idx_flat, table, bsz * seq_p)
    return out.reshape(bsz, seq_p, DP)[:, :seq, : weight.shape[1]]


# speedup vs baseline: 6.0772x; 6.0772x over previous
import functools

import jax
import jax.numpy as jnp
from jax import lax
from jax.experimental import pallas as pl
from jax.experimental.pallas import tpu as pltpu
from jax.experimental.pallas import tpu_sc as plsc

NC, NS = 2, 16
NW = NC * NS
D = 64
DP = 128
CHUNK = 256
NBUF = 4


@functools.partial(jax.jit, static_argnames=("total",))
def _gather(idx2, table2, total):
    b_per_w = total // NW
    n_chunks = b_per_w // CHUNK
    n_waves = n_chunks // NBUF
    mesh = plsc.VectorSubcoreMesh(
        core_axis_name="c", subcore_axis_name="s", num_cores=NC, num_subcores=NS
    )

    @functools.partial(
        pl.kernel,
        mesh=mesh,
        out_type=jax.ShapeDtypeStruct((total, DP), jnp.float32),
        scratch_types=[
            pltpu.VMEM((b_per_w,), jnp.int32),
            pltpu.VMEM((NBUF, CHUNK, D), jnp.float32),
            pltpu.SemaphoreType.DMA,
            pltpu.SemaphoreType.DMA,
            pltpu.SemaphoreType.DMA,
            pltpu.SemaphoreType.DMA,
            pltpu.SemaphoreType.DMA,
            pltpu.SemaphoreType.DMA,
            pltpu.SemaphoreType.DMA,
            pltpu.SemaphoreType.DMA,
        ],
        compiler_params=pltpu.CompilerParams(use_tc_tiling_on_sc=False),
    )
    def kern(idx_hbm, table_hbm, out_hbm, idx_v, rows_v, g0, g1, g2, g3, o0, o1, o2, o3):
        gsem = (g0, g1, g2, g3)
        osem = (o0, o1, o2, o3)
        wid = lax.axis_index("s") * NC + lax.axis_index("c")
        base = wid * b_per_w
        pltpu.sync_copy(idx_hbm.at[pl.ds(base, b_per_w)], idx_v)

        def gather_desc(c, b):
            return pltpu.make_async_copy(
                table_hbm.at[idx_v.at[pl.ds(c * CHUNK, CHUNK)]],
                rows_v.at[b],
                gsem[b],
            )

        def out_desc(c, b):
            return pltpu.make_async_copy(
                rows_v.at[b],
                out_hbm.at[pl.ds(base + c * CHUNK, CHUNK), pl.ds(0, D)],
                osem[b],
            )

        for b in range(NBUF):
            gather_desc(b, b).start()

        @pl.loop(0, n_waves - 1)
        def _(p):
            c = p * NBUF
            for b in range(NBUF):
                gather_desc(c + b, b).wait()
                out_desc(c + b, b).start()
            for b in range(NBUF):
                out_desc(c + b, b).wait()
                gather_desc(c + NBUF + b, b).start()

        c_last = (n_waves - 1) * NBUF
        for b in range(NBUF):
            gather_desc(c_last + b, b).wait()
            out_desc(c_last + b, b).start()
        for b in range(NBUF):
            out_desc(c_last + b, b).wait()

    return kern(idx2, table2)


def kernel(x, weight):
    bsz, seq = x.shape
    n_vocab = weight.shape[0]
    seq_p = 56
    pad_idx = (
        jnp.arange(seq_p - seq, dtype=jnp.int32)[None, :]
        + jnp.arange(bsz, dtype=jnp.int32)[:, None] * 61
    ) % n_vocab
    xp = jnp.concatenate([x.astype(jnp.int32), pad_idx], axis=1)
    idx2 = xp.reshape(bsz * seq_p) * 2
    table2 = jnp.pad(weight, ((0, 0), (0, DP - D))).reshape(2 * n_vocab, D)
    out = _gather(idx2, table2, bsz * seq_p)
    return out.reshape(bsz, seq_p, DP)[:, :seq, :D]
